# speedup vs baseline: 3.3130x; 3.3130x over previous
import functools

import jax
import jax.numpy as jnp
from jax import lax
from jax.experimental import pallas as pl
from jax.experimental.pallas import tpu as pltpu
from jax.experimental.pallas import tpu_sc as plsc

_NC, _NS = 2, 16
_NW = _NC * _NS


@functools.partial(jax.jit, static_argnums=(2, 3))
def _gather_rows(table, idx, B, C):
    V, D = table.shape
    b_per_w = B // _NW
    n_chunks = b_per_w // C
    mesh = plsc.VectorSubcoreMesh(
        core_axis_name="c", subcore_axis_name="s",
        num_cores=_NC, num_subcores=_NS,
    )

    @functools.partial(
        pl.kernel,
        mesh=mesh,
        out_type=jax.ShapeDtypeStruct((B, D), jnp.float32),
        scratch_types=[
            pltpu.VMEM((b_per_w,), jnp.int32),
            pltpu.VMEM((C, D), jnp.float32),
            pltpu.SemaphoreType.DMA,
        ],
    )
    def k(table_hbm, idx_hbm, out_hbm, idx_v, rows_v, sem):
        wid = lax.axis_index("s") * _NC + lax.axis_index("c")
        base = wid * b_per_w
        pltpu.sync_copy(idx_hbm.at[pl.ds(base, b_per_w)], idx_v)

        @pl.loop(0, n_chunks)
        def _(g):
            off = g * C
            pltpu.async_copy(
                table_hbm.at[idx_v.at[pl.ds(off, C)]], rows_v, sem
            ).wait()
            pltpu.sync_copy(rows_v, out_hbm.at[pl.ds(base + off, C)])

    return k(table, idx)


def kernel(x, embedding_matrix):
    B = x.size
    idx = x.reshape(B).astype(jnp.int32)
    out = _gather_rows(embedding_matrix, idx, B, 640)
    return out.reshape(x.shape + (embedding_matrix.shape[1],))


# speedup vs baseline: 3.3379x; 1.0075x over previous
import functools

import jax
import jax.numpy as jnp
from jax import lax
from jax.experimental import pallas as pl
from jax.experimental.pallas import tpu as pltpu
from jax.experimental.pallas import tpu_sc as plsc

_NC, _NS = 2, 16
_NW = _NC * _NS


@functools.partial(jax.jit, static_argnums=(2, 3))
def _gather_rows(table, idx, B, C):
    V, D = table.shape
    b_per_w = B // _NW
    n_chunks = b_per_w // C
    mesh = plsc.VectorSubcoreMesh(
        core_axis_name="c", subcore_axis_name="s",
        num_cores=_NC, num_subcores=_NS,
    )

    @functools.partial(
        pl.kernel,
        mesh=mesh,
        out_type=jax.ShapeDtypeStruct((B, D), jnp.float32),
        scratch_types=[
            pltpu.VMEM((b_per_w,), jnp.int32),
            pltpu.VMEM((C, D), jnp.float32),
            pltpu.VMEM((C, D), jnp.float32),
            pltpu.SemaphoreType.DMA,
            pltpu.SemaphoreType.DMA,
            pltpu.SemaphoreType.DMA,
            pltpu.SemaphoreType.DMA,
        ],
    )
    def k(table_hbm, idx_hbm, out_hbm, idx_v, rows0, rows1,
          g0, g1, s0, s1):
        wid = lax.axis_index("s") * _NC + lax.axis_index("c")
        base = wid * b_per_w
        rows = (rows0, rows1)
        gsem = (g0, g1)
        ssem = (s0, s1)
        pltpu.sync_copy(idx_hbm.at[pl.ds(base, b_per_w)], idx_v)

        def gather(g, b):
            return pltpu.async_copy(
                table_hbm.at[idx_v.at[pl.ds(g * C, C)]], rows[b], gsem[b]
            )

        def store(g, b):
            return pltpu.async_copy(
                rows[b], out_hbm.at[pl.ds(base + g * C, C)], ssem[b]
            )

        def swait(b):
            pltpu.make_async_copy(
                rows[b], out_hbm.at[pl.ds(base, C)], ssem[b]
            ).wait()

        gather(0, 0)
        gather(1, 1)
        for g in range(n_chunks):
            b = g % 2
            pltpu.make_async_copy(
                table_hbm.at[idx_v.at[pl.ds(0, C)]], rows[b], gsem[b]
            ).wait()
            store(g, b)
            if g + 2 < n_chunks:
                swait(b)
                gather(g + 2, b)
        swait(n_chunks % 2)
        swait((n_chunks + 1) % 2)

    return k(table, idx)


def kernel(x, embedding_matrix):
    B = x.size
    idx = x.reshape(B).astype(jnp.int32)
    out = _gather_rows(embedding_matrix, idx, B, 400)
    return out.reshape(x.shape + (embedding_matrix.shape[1],))


# speedup vs baseline: 3.3426x; 1.0014x over previous
import functools

import jax
import jax.numpy as jnp
from jax import lax
from jax.experimental import pallas as pl
from jax.experimental.pallas import tpu as pltpu
from jax.experimental.pallas import tpu_sc as plsc

_NC, _NS = 2, 16
_NW = _NC * _NS


@functools.partial(jax.jit, static_argnums=(2, 3))
def _gather_rows(table, idx, B, C):
    V, D = table.shape
    b_per_w = B // _NW
    n_chunks = b_per_w // C
    mesh = plsc.VectorSubcoreMesh(
        core_axis_name="c", subcore_axis_name="s",
        num_cores=_NC, num_subcores=_NS,
    )

    @functools.partial(
        pl.kernel,
        mesh=mesh,
        out_type=jax.ShapeDtypeStruct((B, D), jnp.float32),
        compiler_params=pltpu.CompilerParams(use_tc_tiling_on_sc=True),
        scratch_types=[
            pltpu.VMEM((b_per_w,), jnp.int32),
            pltpu.VMEM((C, D), jnp.float32),
            pltpu.VMEM((C, D), jnp.float32),
            pltpu.SemaphoreType.DMA,
            pltpu.SemaphoreType.DMA,
            pltpu.SemaphoreType.DMA,
            pltpu.SemaphoreType.DMA,
        ],
    )
    def k(table_hbm, idx_hbm, out_hbm, idx_v, rows0, rows1,
          g0, g1, s0, s1):
        wid = lax.axis_index("s") * _NC + lax.axis_index("c")
        base = wid * b_per_w
        rows = (rows0, rows1)
        gsem = (g0, g1)
        ssem = (s0, s1)
        pltpu.sync_copy(idx_hbm.at[pl.ds(base, b_per_w)], idx_v)

        def gather(g, b):
            return pltpu.async_copy(
                table_hbm.at[idx_v.at[pl.ds(g * C, C)]], rows[b], gsem[b]
            )

        def store(g, b):
            return pltpu.async_copy(
                rows[b], out_hbm.at[pl.ds(base + g * C, C)], ssem[b]
            )

        def swait(b):
            pltpu.make_async_copy(
                rows[b], out_hbm.at[pl.ds(base, C)], ssem[b]
            ).wait()

        gather(0, 0)
        gather(1, 1)
        for g in range(n_chunks):
            b = g % 2
            pltpu.make_async_copy(
                table_hbm.at[idx_v.at[pl.ds(0, C)]], rows[b], gsem[b]
            ).wait()
            store(g, b)
            if g + 2 < n_chunks:
                swait(b)
                gather(g + 2, b)
        swait(n_chunks % 2)
        swait((n_chunks + 1) % 2)

    return k(table, idx)


def kernel(x, embedding_matrix):
    B = x.size
    idx = x.reshape(B).astype(jnp.int32)
    out = _gather_rows(embedding_matrix, idx, B, 400)
    return out.reshape(x.shape + (embedding_matrix.shape[1],))


# speedup vs baseline: 5.7734x; 1.7272x over previous
import functools

import jax
import jax.numpy as jnp
from jax import lax
from jax.experimental import pallas as pl
from jax.experimental.pallas import tpu as pltpu
from jax.experimental.pallas import tpu_sc as plsc

_NC, _NS = 2, 16
_NW = _NC * _NS
_RPC = 8


@jax.jit
def _lookup(table, x):
    V, D = table.shape
    N, S = x.shape
    B = N * S
    b_per_w = B // _NW
    C = _RPC * S
    n_chunks = b_per_w // C
    idx = x.reshape(B)
    mesh = plsc.VectorSubcoreMesh(
        core_axis_name="c", subcore_axis_name="s",
        num_cores=_NC, num_subcores=_NS,
    )

    @functools.partial(
        pl.kernel,
        mesh=mesh,
        out_type=jax.ShapeDtypeStruct((N, S, D), jnp.float32),
        compiler_params=pltpu.CompilerParams(use_tc_tiling_on_sc=True),
        scratch_types=[
            pltpu.VMEM((b_per_w,), jnp.int32),
            pltpu.VMEM((C, D), jnp.float32),
            pltpu.VMEM((C, D), jnp.float32),
            pltpu.SemaphoreType.DMA,
            pltpu.SemaphoreType.DMA,
            pltpu.SemaphoreType.DMA,
            pltpu.SemaphoreType.DMA,
        ],
    )
    def k(table_hbm, idx_hbm, out_hbm, idx_v, rows0, rows1, g0, g1, s0, s1):
        wid = lax.axis_index("s") * _NC + lax.axis_index("c")
        base = wid * b_per_w
        xrow0 = wid * (b_per_w // S)
        rows = (rows0, rows1)
        gsem = (g0, g1)
        ssem = (s0, s1)
        pltpu.sync_copy(idx_hbm.at[pl.ds(base, b_per_w)], idx_v)

        def gather(c, b):
            pltpu.async_copy(
                table_hbm.at[idx_v.at[pl.ds(c * C, C)]], rows[b], gsem[b]
            )

        def gwait(b):
            pltpu.make_async_copy(
                table_hbm.at[idx_v.at[pl.ds(0, C)]], rows[b], gsem[b]
            ).wait()

        def fire_stores(c, b):
            for q in range(_RPC):
                pltpu.async_copy(
                    rows[b].at[pl.ds(q * S, S)],
                    out_hbm.at[xrow0 + c * _RPC + q],
                    ssem[b],
                )

        def drain_stores(b):
            for q in range(_RPC):
                pltpu.make_async_copy(
                    rows[b].at[pl.ds(0, S)], out_hbm.at[xrow0], ssem[b]
                ).wait()

        gather(0, 0)
        gather(1, 1)
        for c in range(n_chunks):
            b = c % 2
            gwait(b)
            fire_stores(c, b)
            if c + 2 < n_chunks:
                drain_stores(b)
                gather(c + 2, b)
        drain_stores(n_chunks % 2)
        drain_stores((n_chunks + 1) % 2)

    return k(table, idx)


def kernel(x, embedding_matrix):
    return _lookup(embedding_matrix, x.astype(jnp.int32))


# speedup vs baseline: 10.3726x; 1.7966x over previous
import functools

import jax
import jax.numpy as jnp
from jax import lax
from jax.experimental import pallas as pl
from jax.experimental.pallas import tpu as pltpu
from jax.experimental.pallas import tpu_sc as plsc

_NC, _NS = 2, 16
_NW = _NC * _NS
_C = 400


@jax.jit
def _lookup(table, x):
    V, D = table.shape
    N, S = x.shape
    B = N * S
    b_per_w = B // _NW
    n_chunks = b_per_w // _C
    idx = jnp.transpose(x).reshape(B)
    mesh = plsc.VectorSubcoreMesh(
        core_axis_name="c", subcore_axis_name="s",
        num_cores=_NC, num_subcores=_NS,
    )

    @functools.partial(
        pl.kernel,
        mesh=mesh,
        out_type=jax.ShapeDtypeStruct((B, D), jnp.float32),
        compiler_params=pltpu.CompilerParams(use_tc_tiling_on_sc=True),
        scratch_types=[
            pltpu.VMEM((b_per_w,), jnp.int32),
            pltpu.VMEM((_C, D), jnp.float32),
            pltpu.VMEM((_C, D), jnp.float32),
            pltpu.SemaphoreType.DMA,
            pltpu.SemaphoreType.DMA,
            pltpu.SemaphoreType.DMA,
            pltpu.SemaphoreType.DMA,
        ],
    )
    def k(table_hbm, idx_hbm, out_hbm, idx_v, rows0, rows1, g0, g1, s0, s1):
        wid = lax.axis_index("s") * _NC + lax.axis_index("c")
        base = wid * b_per_w
        rows = (rows0, rows1)
        gsem = (g0, g1)
        ssem = (s0, s1)
        pltpu.sync_copy(idx_hbm.at[pl.ds(base, b_per_w)], idx_v)

        def gather(c, b):
            pltpu.async_copy(
                table_hbm.at[idx_v.at[pl.ds(c * _C, _C)]], rows[b], gsem[b]
            )

        def gwait(b):
            pltpu.make_async_copy(
                table_hbm.at[idx_v.at[pl.ds(0, _C)]], rows[b], gsem[b]
            ).wait()

        def store(c, b):
            pltpu.async_copy(
                rows[b], out_hbm.at[pl.ds(base + c * _C, _C)], ssem[b]
            )

        def swait(b):
            pltpu.make_async_copy(
                rows[b], out_hbm.at[pl.ds(base, _C)], ssem[b]
            ).wait()

        gather(0, 0)
        gather(1, 1)
        for c in range(n_chunks):
            b = c % 2
            gwait(b)
            store(c, b)
            if c + 2 < n_chunks:
                swait(b)
                gather(c + 2, b)
        swait(n_chunks % 2)
        swait((n_chunks + 1) % 2)

    out = k(table, idx)
    return out.reshape(S, N, D).transpose(1, 0, 2)


def kernel(x, embedding_matrix):
    return _lookup(embedding_matrix, x.astype(jnp.int32))
